# baseline (device time: 338580 ns/iter reference)
import jax
import jax.numpy as jnp
from jax import lax
from jax.experimental import pallas as pl
from jax.experimental.pallas import tpu as pltpu

N_DEV = 16
M = 4096
N = 2048
CH = M // N_DEV

WIRE = jnp.bfloat16
OUT_DTYPE = jnp.bfloat16

_MESH = pl.DeviceIdType.MESH


def kernel(x, w_mat):
    x = x.astype(jnp.bfloat16)
    w = w_mat.astype(jnp.bfloat16)

    def body(x_ref, w_ref, out_ref, buf, qbuf, amax_buf,
             rs_send, rs_recv, ag_send, ag_recv, ax_send, ax_recv):
        p = lax.axis_index("i")
        right = lax.rem(p + 1, N_DEV)
        left = lax.rem(p + N_DEV - 1, N_DEV)

        def partial(c):
            xs = x_ref[pl.ds(c * CH, CH), :]
            return jnp.dot(xs, w_ref[...], preferred_element_type=jnp.float32)

        bsem = pltpu.get_barrier_semaphore()
        pl.semaphore_signal(bsem, inc=1, device_id=(left,), device_id_type=_MESH)
        pl.semaphore_signal(bsem, inc=1, device_id=(right,), device_id_type=_MESH)
        pl.semaphore_wait(bsem, 2)

        buf[0, :, :] = partial(p).astype(WIRE)
        for h in range(N_DEV - 1):
            c = lax.rem(p + N_DEV - h - 1, N_DEV)
            rdma = pltpu.make_async_remote_copy(
                src_ref=buf.at[h],
                dst_ref=buf.at[h + 1],
                send_sem=rs_send.at[h],
                recv_sem=rs_recv.at[h],
                device_id=(right,),
                device_id_type=_MESH,
            )
            rdma.start()
            pc = partial(c)
            rdma.wait()
            buf[h + 1, :, :] = (buf[h + 1, :, :].astype(jnp.float32) + pc).astype(WIRE)

        o = lax.rem(p + 1, N_DEV)
        r = jnp.maximum(buf[N_DEV - 1, :, :].astype(jnp.float32), 0.0)
        amax_local = jnp.max(r)

        amax_buf[p, :, :] = jnp.full((8, 128), amax_local, jnp.float32)
        sends = []
        for d in range(1, N_DEV):
            t = lax.rem(p + d, N_DEV)
            rd = pltpu.make_async_remote_copy(
                src_ref=amax_buf.at[p],
                dst_ref=amax_buf.at[p],
                send_sem=ax_send.at[d - 1],
                recv_sem=ax_recv.at[d - 1],
                device_id=(t,),
                device_id_type=_MESH,
            )
            rd.start()
            sends.append(rd)
        for d in range(1, N_DEV):
            s = lax.rem(p + N_DEV - d, N_DEV)
            wd = pltpu.make_async_remote_copy(
                src_ref=amax_buf.at[p],
                dst_ref=amax_buf.at[s],
                send_sem=ax_send.at[d - 1],
                recv_sem=ax_recv.at[d - 1],
                device_id=(s,),
                device_id_type=_MESH,
            )
            wd.wait_recv()
        for rd in sends:
            rd.wait_send()

        amax_g = jnp.max(amax_buf[...])
        scale = amax_g / 127.0
        inv_scale = 127.0 / amax_g
        q = jnp.clip(
            lax.round(r * inv_scale, lax.RoundingMethod.TO_NEAREST_EVEN),
            0.0, 127.0,
        ).astype(jnp.int8)
        qbuf[o, :, :] = q
        out_ref[pl.ds(o * CH, CH), :] = (q.astype(jnp.float32) * scale).astype(OUT_DTYPE)

        for g in range(N_DEV - 1):
            c_s = lax.rem(o + N_DEV - g, N_DEV)
            c_r = lax.rem(o + N_DEV - g - 1, N_DEV)
            rdma = pltpu.make_async_remote_copy(
                src_ref=qbuf.at[c_s],
                dst_ref=qbuf.at[c_s],
                send_sem=ag_send.at[g],
                recv_sem=ag_recv.at[g],
                device_id=(right,),
                device_id_type=_MESH,
            )
            rdma.start()
            rdma.wait()
            out_ref[pl.ds(c_r * CH, CH), :] = (
                qbuf[c_r, :, :].astype(jnp.float32) * scale
            ).astype(OUT_DTYPE)

    try:
        cparams = pltpu.CompilerParams(collective_id=0)
    except AttributeError:
        cparams = pltpu.TPUCompilerParams(collective_id=0)

    return pl.pallas_call(
        body,
        out_shape=jax.ShapeDtypeStruct((M, N), OUT_DTYPE),
        in_specs=[
            pl.BlockSpec(memory_space=pltpu.VMEM),
            pl.BlockSpec(memory_space=pltpu.VMEM),
        ],
        out_specs=pl.BlockSpec(memory_space=pltpu.VMEM),
        scratch_shapes=[
            pltpu.VMEM((N_DEV, CH, N), WIRE),
            pltpu.VMEM((N_DEV, CH, N), jnp.int8),
            pltpu.VMEM((N_DEV, 8, 128), jnp.float32),
            pltpu.SemaphoreType.DMA((N_DEV - 1,)),
            pltpu.SemaphoreType.DMA((N_DEV - 1,)),
            pltpu.SemaphoreType.DMA((N_DEV - 1,)),
            pltpu.SemaphoreType.DMA((N_DEV - 1,)),
            pltpu.SemaphoreType.DMA((N_DEV - 1,)),
            pltpu.SemaphoreType.DMA((N_DEV - 1,)),
        ],
        compiler_params=cparams,
    )(x, w)


# device time: 280633 ns/iter; 1.2065x vs baseline; 1.2065x over previous
import jax
import jax.numpy as jnp
from jax import lax
from jax.experimental import pallas as pl
from jax.experimental.pallas import tpu as pltpu

N_DEV = 16
M = 4096
N = 2048
CH = M // N_DEV
HCH = CH // 2

WIRE = jnp.bfloat16
OUT_DTYPE = jnp.bfloat16

_MESH = pl.DeviceIdType.MESH


def kernel(x, w_mat):
    x = x.astype(jnp.bfloat16)
    w = w_mat.astype(jnp.bfloat16)

    def body(x_ref, w_ref, out_ref, buf, qbuf, amax_buf,
             rs_send, rs_recv, ag_send, ag_recv, ax_send, ax_recv):
        p = lax.axis_index("i")
        right = lax.rem(p + 1, N_DEV)
        left = lax.rem(p + N_DEV - 1, N_DEV)

        def partial(c):
            xs = x_ref[pl.ds(c * CH, CH), :]
            return jnp.dot(xs, w_ref[...], preferred_element_type=jnp.float32)

        def rs_rdma(h, half):
            sl = pl.ds(half * HCH, HCH)
            return pltpu.make_async_remote_copy(
                src_ref=buf.at[h, sl, :],
                dst_ref=buf.at[h + 1, sl, :],
                send_sem=rs_send.at[h, half],
                recv_sem=rs_recv.at[h, half],
                device_id=(right,),
                device_id_type=_MESH,
            )

        bsem = pltpu.get_barrier_semaphore()
        pl.semaphore_signal(bsem, inc=1, device_id=(left,), device_id_type=_MESH)
        pl.semaphore_signal(bsem, inc=1, device_id=(right,), device_id_type=_MESH)
        pl.semaphore_wait(bsem, 2)

        buf[0, :, :] = partial(p).astype(WIRE)
        d0 = rs_rdma(0, 0)
        d1 = rs_rdma(0, 1)
        d0.start()
        d1.start()
        r = None
        for h in range(N_DEV - 1):
            c = lax.rem(p + N_DEV - h - 1, N_DEV)
            pc = partial(c)
            last = h == N_DEV - 2
            d0.wait()
            if not last:
                buf[h + 1, :HCH, :] = (
                    buf[h + 1, :HCH, :].astype(jnp.float32) + pc[:HCH, :]
                ).astype(WIRE)
                n0 = rs_rdma(h + 1, 0)
                n0.start()
                d1.wait()
                buf[h + 1, HCH:, :] = (
                    buf[h + 1, HCH:, :].astype(jnp.float32) + pc[HCH:, :]
                ).astype(WIRE)
                n1 = rs_rdma(h + 1, 1)
                n1.start()
                d0, d1 = n0, n1
            else:
                top = buf[h + 1, :HCH, :].astype(jnp.float32) + pc[:HCH, :]
                d1.wait()
                bot = buf[h + 1, HCH:, :].astype(jnp.float32) + pc[HCH:, :]
                r = jnp.maximum(jnp.concatenate([top, bot], axis=0), 0.0)

        o = lax.rem(p + 1, N_DEV)
        amax_local = jnp.max(r)

        amax_buf[p, :, :] = jnp.full((8, 128), amax_local, jnp.float32)
        sends = []
        for d in range(1, N_DEV):
            t = lax.rem(p + d, N_DEV)
            rd = pltpu.make_async_remote_copy(
                src_ref=amax_buf.at[p],
                dst_ref=amax_buf.at[p],
                send_sem=ax_send.at[d - 1],
                recv_sem=ax_recv.at[d - 1],
                device_id=(t,),
                device_id_type=_MESH,
            )
            rd.start()
            sends.append(rd)
        for d in range(1, N_DEV):
            s = lax.rem(p + N_DEV - d, N_DEV)
            wd = pltpu.make_async_remote_copy(
                src_ref=amax_buf.at[p],
                dst_ref=amax_buf.at[s],
                send_sem=ax_send.at[d - 1],
                recv_sem=ax_recv.at[d - 1],
                device_id=(s,),
                device_id_type=_MESH,
            )
            wd.wait_recv()
        for rd in sends:
            rd.wait_send()

        amax_g = jnp.max(amax_buf[...])
        scale = amax_g / 127.0
        inv_scale = 127.0 / amax_g
        q = jnp.clip(
            lax.round(r * inv_scale, lax.RoundingMethod.TO_NEAREST_EVEN),
            0.0, 127.0,
        ).astype(jnp.int8)
        qbuf[o, :, :] = q

        def ag_rdma(g, chunk, half):
            sl = pl.ds(half * HCH, HCH)
            return pltpu.make_async_remote_copy(
                src_ref=qbuf.at[chunk, sl, :],
                dst_ref=qbuf.at[chunk, sl, :],
                send_sem=ag_send.at[g, half],
                recv_sem=ag_recv.at[g, half],
                device_id=(right,),
                device_id_type=_MESH,
            )

        def dequant(c):
            out_ref[pl.ds(c * CH, CH), :] = (
                qbuf[c, :, :].astype(jnp.float32) * scale
            ).astype(OUT_DTYPE)

        a0 = ag_rdma(0, o, 0)
        a1 = ag_rdma(0, o, 1)
        a0.start()
        a1.start()
        dequant(o)
        for g in range(N_DEV - 1):
            c_r = lax.rem(o + N_DEV - g - 1, N_DEV)
            a0.wait()
            if g < N_DEV - 2:
                n0 = ag_rdma(g + 1, c_r, 0)
                n0.start()
                a1.wait()
                n1 = ag_rdma(g + 1, c_r, 1)
                n1.start()
                a0, a1 = n0, n1
            else:
                a1.wait()
            dequant(c_r)

    try:
        cparams = pltpu.CompilerParams(collective_id=0)
    except AttributeError:
        cparams = pltpu.TPUCompilerParams(collective_id=0)

    return pl.pallas_call(
        body,
        out_shape=jax.ShapeDtypeStruct((M, N), OUT_DTYPE),
        in_specs=[
            pl.BlockSpec(memory_space=pltpu.VMEM),
            pl.BlockSpec(memory_space=pltpu.VMEM),
        ],
        out_specs=pl.BlockSpec(memory_space=pltpu.VMEM),
        scratch_shapes=[
            pltpu.VMEM((N_DEV, CH, N), WIRE),
            pltpu.VMEM((N_DEV, CH, N), jnp.int8),
            pltpu.VMEM((N_DEV, 8, 128), jnp.float32),
            pltpu.SemaphoreType.DMA((N_DEV - 1, 2)),
            pltpu.SemaphoreType.DMA((N_DEV - 1, 2)),
            pltpu.SemaphoreType.DMA((N_DEV - 1, 2)),
            pltpu.SemaphoreType.DMA((N_DEV - 1, 2)),
            pltpu.SemaphoreType.DMA((N_DEV - 1,)),
            pltpu.SemaphoreType.DMA((N_DEV - 1,)),
        ],
        compiler_params=cparams,
    )(x, w)
